# baseline (speedup 1.0000x reference)
import functools

import jax
import jax.numpy as jnp
from jax import lax
from jax.experimental import pallas as pl
from jax.experimental.pallas import tpu as pltpu
from jax.experimental.pallas import tpu_sc as plsc

K_NEURONS = 10000
FEAT = 784
FEAT_PAD = 896
BATCH = 4096

BB = 512
KB = 2048
NKB = 5


def _bmu_kernel(xt_ref, wt_ref, xsq_ref, wsq_ref, idx_out, min_scr, idx_scr):
    k = pl.program_id(0)
    b = pl.program_id(1)
    nk = pl.num_programs(0)

    x_sq = xsq_ref[...]
    w_sq = wsq_ref[...]
    dot2 = lax.dot_general(xt_ref[...] * 2.0, wt_ref[...],
                           (((0,), (0,)), ((), ())),
                           preferred_element_type=jnp.float32)
    s = x_sq - dot2 + w_sq

    cols = lax.broadcasted_iota(jnp.int32, (BB, KB), 1) + k * KB
    s = jnp.where(cols < K_NEURONS, s, jnp.inf)
    m = jnp.min(s, axis=1, keepdims=True)
    idx = jnp.min(jnp.where(s == m, cols, jnp.int32(2**30)),
                  axis=1, keepdims=True)

    rows = pl.ds(b * BB, BB)

    @pl.when(k == 0)
    def _():
        min_scr[rows, :] = m
        idx_scr[rows, :] = idx

    @pl.when(k > 0)
    def _():
        prev_m = min_scr[rows, :]
        prev_i = idx_scr[rows, :]
        better = m < prev_m
        min_scr[rows, :] = jnp.where(better, m, prev_m)
        idx_scr[rows, :] = jnp.where(better, idx, prev_i)

    @pl.when(k == nk - 1)
    def _():
        idx_out[...] = jnp.reshape(idx_scr[rows, :], (BB,))


def _bmu_indices(xt, wt, x_sq, w_sq):
    grid = (NKB, BATCH // BB)
    return pl.pallas_call(
        _bmu_kernel,
        grid=grid,
        in_specs=[
            pl.BlockSpec((FEAT, BB), lambda k, b: (0, b)),
            pl.BlockSpec((FEAT, KB), lambda k, b: (0, k)),
            pl.BlockSpec((BB, 1), lambda k, b: (b, 0)),
            pl.BlockSpec((1, KB), lambda k, b: (0, k)),
        ],
        out_specs=pl.BlockSpec((BB,), lambda k, b: (b,)),
        out_shape=jax.ShapeDtypeStruct((BATCH,), jnp.int32),
        scratch_shapes=[
            pltpu.VMEM((BATCH, 1), jnp.float32),
            pltpu.VMEM((BATCH, 1), jnp.int32),
        ],
        compiler_params=pltpu.CompilerParams(
            dimension_semantics=("arbitrary", "arbitrary")),
    )(xt, wt, x_sq, w_sq)


def _make_sc_gather():
    info = plsc.get_sparse_core_info()
    nc, ns = info.num_cores, info.num_subcores
    nw = nc * ns
    b_per_w = BATCH // nw
    mesh = plsc.VectorSubcoreMesh(core_axis_name="c", subcore_axis_name="s")

    @functools.partial(
        pl.kernel, mesh=mesh,
        compiler_params=pltpu.CompilerParams(use_tc_tiling_on_sc=False),
        out_type=jax.ShapeDtypeStruct((BATCH, FEAT), jnp.float32),
        scratch_types=[
            pltpu.VMEM((b_per_w,), jnp.int32),
            pltpu.VMEM((b_per_w, FEAT), jnp.float32),
            pltpu.SemaphoreType.DMA,
        ],
    )
    def gather(table_hbm, idx_hbm, out_hbm, idx_v, rows_v, sem):
        wid = lax.axis_index("s") * nc + lax.axis_index("c")
        base = wid * b_per_w
        pltpu.sync_copy(idx_hbm.at[pl.ds(base, b_per_w)], idx_v)
        pltpu.async_copy(table_hbm.at[idx_v], rows_v, sem).wait()
        pltpu.sync_copy(rows_v, out_hbm.at[pl.ds(base, b_per_w)])

    return gather


_sc_gather = None


def kernel(inputs, weights):
    global _sc_gather
    if _sc_gather is None:
        _sc_gather = _make_sc_gather()
    x = jnp.reshape(inputs, (BATCH, FEAT))
    x_sq = jnp.sum(x * x, axis=1, keepdims=True)
    w_sq = jnp.sum(weights * weights, axis=1)[None, :]
    idx = _bmu_indices(x.T, weights.T, x_sq, w_sq)
    return _sc_gather(weights, idx)


# speedup vs baseline: 1.1513x; 1.1513x over previous
import functools

import jax
import jax.numpy as jnp
from jax import lax
from jax.experimental import pallas as pl
from jax.experimental.pallas import tpu as pltpu
from jax.experimental.pallas import tpu_sc as plsc

K_NEURONS = 10000
FEAT = 784
FEAT_PAD = 896
BATCH = 4096

BB = 512
KB = 2560
NKB = 4


def _bmu_kernel(xt_ref, wt_ref, xsq_ref, wsq_ref, idx_out, min_scr, idx_scr):
    k = pl.program_id(0)
    b = pl.program_id(1)
    nk = pl.num_programs(0)

    x_sq = xsq_ref[...]
    w_sq = wsq_ref[...]
    dot2 = lax.dot_general(xt_ref[...] * 2.0, wt_ref[...],
                           (((0,), (0,)), ((), ())),
                           preferred_element_type=jnp.float32)
    s = x_sq - dot2 + w_sq

    cols = lax.broadcasted_iota(jnp.int32, (BB, KB), 1) + k * KB
    s = jnp.where(cols < K_NEURONS, s, jnp.inf)
    m = jnp.min(s, axis=1, keepdims=True)
    idx = jnp.min(jnp.where(s == m, cols, jnp.int32(2**30)),
                  axis=1, keepdims=True)

    rows = pl.ds(b * BB, BB)

    @pl.when(k == 0)
    def _():
        min_scr[rows, :] = m
        idx_scr[rows, :] = idx

    @pl.when(k > 0)
    def _():
        prev_m = min_scr[rows, :]
        prev_i = idx_scr[rows, :]
        better = m < prev_m
        min_scr[rows, :] = jnp.where(better, m, prev_m)
        idx_scr[rows, :] = jnp.where(better, idx, prev_i)

    @pl.when(k == nk - 1)
    def _():
        idx_out[...] = jnp.reshape(idx_scr[rows, :], (BB,))


def _bmu_indices(xt, wt, x_sq, w_sq):
    grid = (NKB, BATCH // BB)
    return pl.pallas_call(
        _bmu_kernel,
        grid=grid,
        in_specs=[
            pl.BlockSpec((FEAT, BB), lambda k, b: (0, b)),
            pl.BlockSpec((FEAT, KB), lambda k, b: (0, k)),
            pl.BlockSpec((BB, 1), lambda k, b: (b, 0)),
            pl.BlockSpec((1, KB), lambda k, b: (0, k)),
        ],
        out_specs=pl.BlockSpec((BB,), lambda k, b: (b,)),
        out_shape=jax.ShapeDtypeStruct((BATCH,), jnp.int32),
        scratch_shapes=[
            pltpu.VMEM((BATCH, 1), jnp.float32),
            pltpu.VMEM((BATCH, 1), jnp.int32),
        ],
        compiler_params=pltpu.CompilerParams(
            dimension_semantics=("arbitrary", "arbitrary")),
    )(xt, wt, x_sq, w_sq)


def _make_sc_gather():
    info = plsc.get_sparse_core_info()
    nc, ns = info.num_cores, info.num_subcores
    nw = nc * ns
    b_per_w = BATCH // nw
    mesh = plsc.VectorSubcoreMesh(core_axis_name="c", subcore_axis_name="s")

    @functools.partial(
        pl.kernel, mesh=mesh,
        out_type=jax.ShapeDtypeStruct((BATCH, FEAT_PAD), jnp.float32),
        scratch_types=[
            pltpu.VMEM((b_per_w,), jnp.int32),
            pltpu.VMEM((b_per_w, FEAT_PAD), jnp.float32),
            pltpu.SemaphoreType.DMA,
        ],
    )
    def gather(table_hbm, idx_hbm, out_hbm, idx_v, rows_v, sem):
        wid = lax.axis_index("s") * nc + lax.axis_index("c")
        base = wid * b_per_w
        pltpu.sync_copy(idx_hbm.at[pl.ds(base, b_per_w)], idx_v)
        pltpu.async_copy(table_hbm.at[idx_v], rows_v, sem).wait()
        pltpu.sync_copy(rows_v, out_hbm.at[pl.ds(base, b_per_w)])

    return gather


_sc_gather = None


def kernel(inputs, weights):
    global _sc_gather
    if _sc_gather is None:
        _sc_gather = _make_sc_gather()
    x = jnp.reshape(inputs, (BATCH, FEAT))
    x_sq = jnp.sum(x * x, axis=1, keepdims=True)
    w_sq = jnp.sum(weights * weights, axis=1)[None, :]
    w896 = jnp.pad(weights, ((0, 0), (0, FEAT_PAD - FEAT)))
    idx = _bmu_indices(x.T, weights.T, x_sq, w_sq)
    bmu896 = _sc_gather(w896, idx)
    return bmu896[:, :FEAT]


# speedup vs baseline: 1.2786x; 1.1106x over previous
import functools

import jax
import jax.numpy as jnp
from jax import lax
from jax.experimental import pallas as pl
from jax.experimental.pallas import tpu as pltpu
from jax.experimental.pallas import tpu_sc as plsc

K_NEURONS = 10000
FEAT = 784
FEAT_PAD = 896
BATCH = 4096

BB = 512
KB = 2560
NKB = 4


def _bmu_kernel(xt_ref, wt_ref, xsq_ref, wsq_ref, idx_out, w896_out,
                min_scr, idx_scr):
    k = pl.program_id(0)
    b = pl.program_id(1)
    nk = pl.num_programs(0)

    x_sq = xsq_ref[...]
    w_sq = wsq_ref[...]

    @pl.when(b == 0)
    def _():
        w896_out[:, :FEAT] = jnp.transpose(wt_ref[...])
    dot2 = lax.dot_general(xt_ref[...] * 2.0, wt_ref[...],
                           (((0,), (0,)), ((), ())),
                           preferred_element_type=jnp.float32)
    s = x_sq - dot2 + w_sq

    cols = lax.broadcasted_iota(jnp.int32, (BB, KB), 1) + k * KB
    s = jnp.where(cols < K_NEURONS, s, jnp.inf)
    m = jnp.min(s, axis=1, keepdims=True)
    idx = jnp.min(jnp.where(s == m, cols, jnp.int32(2**30)),
                  axis=1, keepdims=True)

    rows = pl.ds(b * BB, BB)

    @pl.when(k == 0)
    def _():
        min_scr[rows, :] = m
        idx_scr[rows, :] = idx

    @pl.when(k > 0)
    def _():
        prev_m = min_scr[rows, :]
        prev_i = idx_scr[rows, :]
        better = m < prev_m
        min_scr[rows, :] = jnp.where(better, m, prev_m)
        idx_scr[rows, :] = jnp.where(better, idx, prev_i)

    @pl.when(k == nk - 1)
    def _():
        idx_out[...] = jnp.reshape(idx_scr[rows, :], (BB,))


def _bmu_indices(xt, wt, x_sq, w_sq):
    grid = (NKB, BATCH // BB)
    return pl.pallas_call(
        _bmu_kernel,
        grid=grid,
        in_specs=[
            pl.BlockSpec((FEAT, BB), lambda k, b: (0, b)),
            pl.BlockSpec((FEAT, KB), lambda k, b: (0, k)),
            pl.BlockSpec((BB, 1), lambda k, b: (b, 0)),
            pl.BlockSpec((1, KB), lambda k, b: (0, k)),
        ],
        out_specs=[
            pl.BlockSpec((BB,), lambda k, b: (b,)),
            pl.BlockSpec((KB, FEAT_PAD), lambda k, b: (k, 0)),
        ],
        out_shape=[
            jax.ShapeDtypeStruct((BATCH,), jnp.int32),
            jax.ShapeDtypeStruct((NKB * KB, FEAT_PAD), jnp.float32),
        ],
        scratch_shapes=[
            pltpu.VMEM((BATCH, 1), jnp.float32),
            pltpu.VMEM((BATCH, 1), jnp.int32),
        ],
        compiler_params=pltpu.CompilerParams(
            dimension_semantics=("arbitrary", "arbitrary")),
    )(xt, wt, x_sq, w_sq)


def _make_sc_gather():
    info = plsc.get_sparse_core_info()
    nc, ns = info.num_cores, info.num_subcores
    nw = nc * ns
    b_per_w = BATCH // nw
    mesh = plsc.VectorSubcoreMesh(core_axis_name="c", subcore_axis_name="s")

    @functools.partial(
        pl.kernel, mesh=mesh,
        out_type=jax.ShapeDtypeStruct((BATCH, FEAT_PAD), jnp.float32),
        scratch_types=[
            pltpu.VMEM((b_per_w,), jnp.int32),
            pltpu.VMEM((b_per_w, FEAT_PAD), jnp.float32),
            pltpu.SemaphoreType.DMA,
        ],
    )
    def gather(table_hbm, idx_hbm, out_hbm, idx_v, rows_v, sem):
        wid = lax.axis_index("s") * nc + lax.axis_index("c")
        base = wid * b_per_w
        pltpu.sync_copy(idx_hbm.at[pl.ds(base, b_per_w)], idx_v)
        pltpu.async_copy(table_hbm.at[idx_v], rows_v, sem).wait()
        pltpu.sync_copy(rows_v, out_hbm.at[pl.ds(base, b_per_w)])

    return gather


_sc_gather = None


def kernel(inputs, weights):
    global _sc_gather
    if _sc_gather is None:
        _sc_gather = _make_sc_gather()
    x = jnp.reshape(inputs, (BATCH, FEAT))
    x_sq = jnp.sum(x * x, axis=1, keepdims=True)
    w_sq = jnp.sum(weights * weights, axis=1)[None, :]
    idx, w896 = _bmu_indices(x.T, weights.T, x_sq, w_sq)
    bmu896 = _sc_gather(w896, idx)
    return bmu896[:, :FEAT]


# speedup vs baseline: 1.3332x; 1.0427x over previous
import functools

import jax
import jax.numpy as jnp
from jax import lax
from jax.experimental import pallas as pl
from jax.experimental.pallas import tpu as pltpu
from jax.experimental.pallas import tpu_sc as plsc

K_NEURONS = 10000
FEAT = 784
FEAT_PAD = 896
BATCH = 4096

BB = 1024
KB = 2560
NKB = 4


def _bmu_kernel(xt_ref, wt_ref, xsq_ref, wsq_ref, idx_out, w896_out,
                min_scr, idx_scr):
    k = pl.program_id(0)
    b = pl.program_id(1)
    nk = pl.num_programs(0)

    x_sq = xsq_ref[...]
    w_sq = wsq_ref[...]

    @pl.when(b == 0)
    def _():
        w896_out[:, :FEAT] = jnp.transpose(wt_ref[...])
    dot2 = lax.dot_general(xt_ref[...] * 2.0, wt_ref[...],
                           (((0,), (0,)), ((), ())),
                           preferred_element_type=jnp.float32)
    s = x_sq - dot2 + w_sq

    cols = lax.broadcasted_iota(jnp.int32, (BB, KB), 1) + k * KB
    s = jnp.where(cols < K_NEURONS, s, jnp.inf)
    m = jnp.min(s, axis=1, keepdims=True)
    idx = jnp.min(jnp.where(s == m, cols, jnp.int32(2**30)),
                  axis=1, keepdims=True)

    rows = pl.ds(b * BB, BB)

    @pl.when(k == 0)
    def _():
        min_scr[rows, :] = m
        idx_scr[rows, :] = idx

    @pl.when(k > 0)
    def _():
        prev_m = min_scr[rows, :]
        prev_i = idx_scr[rows, :]
        better = m < prev_m
        min_scr[rows, :] = jnp.where(better, m, prev_m)
        idx_scr[rows, :] = jnp.where(better, idx, prev_i)

    @pl.when(k == nk - 1)
    def _():
        idx_out[...] = jnp.reshape(idx_scr[rows, :], (BB,))


def _bmu_indices(xt, wt, x_sq, w_sq):
    grid = (NKB, BATCH // BB)
    return pl.pallas_call(
        _bmu_kernel,
        grid=grid,
        in_specs=[
            pl.BlockSpec((FEAT, BB), lambda k, b: (0, b)),
            pl.BlockSpec((FEAT, KB), lambda k, b: (0, k)),
            pl.BlockSpec((BB, 1), lambda k, b: (b, 0)),
            pl.BlockSpec((1, KB), lambda k, b: (0, k)),
        ],
        out_specs=[
            pl.BlockSpec((BB,), lambda k, b: (b,)),
            pl.BlockSpec((KB, FEAT_PAD), lambda k, b: (k, 0)),
        ],
        out_shape=[
            jax.ShapeDtypeStruct((BATCH,), jnp.int32),
            jax.ShapeDtypeStruct((NKB * KB, FEAT_PAD), jnp.float32),
        ],
        scratch_shapes=[
            pltpu.VMEM((BATCH, 1), jnp.float32),
            pltpu.VMEM((BATCH, 1), jnp.int32),
        ],
        compiler_params=pltpu.CompilerParams(
            dimension_semantics=("arbitrary", "arbitrary")),
    )(xt, wt, x_sq, w_sq)


def _make_sc_gather():
    info = plsc.get_sparse_core_info()
    nc, ns = info.num_cores, info.num_subcores
    nw = nc * ns
    b_per_w = BATCH // nw
    mesh = plsc.VectorSubcoreMesh(core_axis_name="c", subcore_axis_name="s")

    @functools.partial(
        pl.kernel, mesh=mesh,
        out_type=jax.ShapeDtypeStruct((BATCH, FEAT_PAD), jnp.float32),
        scratch_types=[
            pltpu.VMEM((b_per_w,), jnp.int32),
            pltpu.VMEM((b_per_w, FEAT_PAD), jnp.float32),
            pltpu.SemaphoreType.DMA,
        ],
    )
    def gather(table_hbm, idx_hbm, out_hbm, idx_v, rows_v, sem):
        wid = lax.axis_index("s") * nc + lax.axis_index("c")
        base = wid * b_per_w
        pltpu.sync_copy(idx_hbm.at[pl.ds(base, b_per_w)], idx_v)
        pltpu.async_copy(table_hbm.at[idx_v], rows_v, sem).wait()
        pltpu.sync_copy(rows_v, out_hbm.at[pl.ds(base, b_per_w)])

    return gather


_sc_gather = None


def kernel(inputs, weights):
    global _sc_gather
    if _sc_gather is None:
        _sc_gather = _make_sc_gather()
    x = jnp.reshape(inputs, (BATCH, FEAT))
    x_sq = jnp.sum(x * x, axis=1, keepdims=True)
    w_sq = jnp.sum(weights * weights, axis=1)[None, :]
    idx, w896 = _bmu_indices(x.T, weights.T, x_sq, w_sq)
    bmu896 = _sc_gather(w896, idx)
    return bmu896[:, :FEAT]
